# Initial kernel scaffold; baseline (speedup 1.0000x reference)
import jax
import jax.numpy as jnp
from jax.experimental import pallas as pl


def kernel(node_deg, node_lab, pos, edge_index, embed_deg, embed_lab, eps1, W1, b1, g1, be1, eps2, W2, b2, g2, be2, Wf1, bf1, gf, bef, Wf2, bf2):
    raise NotImplementedError("write your pallas kernel here")


# speedup vs baseline: 10.6220x; 10.6220x over previous
import functools

import jax
import jax.numpy as jnp
from jax import lax
from jax.experimental import pallas as pl
from jax.experimental.pallas import tpu as pltpu
from jax.experimental.pallas import tpu_sc as plsc

_NC = 2
_NS = 16
_NT = _NC * _NS
_CH = 128


def _make_edge_agg(np_, epw, nchunk, d):
    mesh = plsc.VectorSubcoreMesh(core_axis_name="c", subcore_axis_name="s")

    @functools.partial(
        pl.kernel,
        out_type=jax.ShapeDtypeStruct((_NC, np_, d), jnp.float32),
        mesh=mesh,
        compiler_params=pltpu.CompilerParams(use_tc_tiling_on_sc=(d % 128 == 0)),
        scratch_types=[
            pltpu.VMEM((epw,), jnp.int32),
            pltpu.VMEM((_CH,), jnp.int32),
            pltpu.VMEM((_CH,), jnp.int32),
            pltpu.VMEM((_CH, d), jnp.float32),
            pltpu.VMEM((_CH, d), jnp.float32),
            pltpu.VMEM_SHARED((np_, d), jnp.float32),
            pltpu.SemaphoreType.DMA,
            pltpu.SemaphoreType.DMA,
            pltpu.SemaphoreType.DMA,
            pltpu.SemaphoreType.DMA,
            pltpu.SemaphoreType.DMA,
            pltpu.SemaphoreType.DMA,
        ],
    )
    def agg(x_hbm, src_hbm, dst_hbm, zero_hbm, out_hbm,
            src_v, didx0, didx1, rows0, rows1, agg_sh,
            g0, g1, s0, s1, d0, d1):
        c = lax.axis_index("c")
        s = lax.axis_index("s")
        wid = c * _NS + s

        @pl.when(s == 0)
        def _():
            pltpu.sync_copy(zero_hbm, agg_sh)

        pltpu.sync_copy(src_hbm.at[wid], src_v)

        didx = (didx0, didx1)
        rows = (rows0, rows1)
        gsem = (g0, g1)
        ssem = (s0, s1)
        dsem = (d0, d1)

        def dload(i, b):
            return pltpu.async_copy(dst_hbm.at[wid, i], didx[b], dsem[b])

        def gather(i, b):
            return pltpu.async_copy(
                x_hbm.at[src_v.at[pl.ds(i * _CH, _CH)]], rows[b], gsem[b])

        def scat(b):
            return pltpu.async_copy(
                rows[b], agg_sh.at[didx[b]], ssem[b], add=True)

        gd = {0: gather(0, 0)}
        dd = {0: dload(0, 0)}
        plsc.subcore_barrier()

        sd = {}
        for i in range(nchunk):
            b = i % 2
            gd[i].wait()
            dd[i].wait()
            sd[i] = scat(b)
            if i + 1 < nchunk:
                if i >= 1:
                    sd[i - 1].wait()
                dd[i + 1] = dload(i + 1, 1 - b)
                gd[i + 1] = gather(i + 1, 1 - b)
        sd[nchunk - 1].wait()
        sd[nchunk - 2].wait()

        plsc.subcore_barrier()

        for cc in range(_NC):
            @pl.when((s == 0) & (c == cc))
            def _():
                pltpu.sync_copy(agg_sh, out_hbm.at[cc])

    return agg


def _lrelu(x):
    return jnp.where(x >= 0, x, 0.01 * x)


def _bn(y, g, b):
    m = jnp.mean(y, axis=0, keepdims=True)
    v = jnp.mean((y - m) * (y - m), axis=0, keepdims=True)
    return g * (y - m) * lax.rsqrt(v + 1e-5) + b


def _make_ctrb(n):
    def body(idx_ref, ed_ref, el_ref, wfa_ref, wfb_ref, bf1_ref, out_ref):
        it = lax.broadcasted_iota(jnp.int32, (n, 128), 1)
        ohd = (idx_ref[:, 0:1] == it).astype(jnp.float32)
        ohl = (idx_ref[:, 1:2] == it).astype(jnp.float32)
        edw = jnp.dot(ed_ref[...], wfa_ref[...],
                      preferred_element_type=jnp.float32)
        elw = jnp.dot(el_ref[...], wfb_ref[...],
                      preferred_element_type=jnp.float32)
        out_ref[...] = (
            jnp.dot(ohd, edw, preferred_element_type=jnp.float32)
            + jnp.dot(ohl, elw, preferred_element_type=jnp.float32)
            + bf1_ref[...])

    return pl.pallas_call(
        body, out_shape=jax.ShapeDtypeStruct((n, 128), jnp.float32))


def _make_layer1(n, np_):
    def body(pos_ref, agg_ref, w1_ref, b1_ref, g1_ref, be1_ref, eps_ref,
             x1_ref):
        u = ((1.0 + eps_ref[0, 0]) * pos_ref[:n, :]
             + agg_ref[0, :n, :] + agg_ref[1, :n, :])
        y = jnp.dot(u, w1_ref[...], preferred_element_type=jnp.float32)
        y = y + b1_ref[...]
        x1_ref[:n, :] = _lrelu(_bn(y, g1_ref[...], be1_ref[...]))
        x1_ref[n:, :] = jnp.zeros((np_ - n, 128), jnp.float32)

    return pl.pallas_call(
        body, out_shape=jax.ShapeDtypeStruct((np_, 128), jnp.float32))


def _make_tail(n):
    def body(x1_ref, agg_ref, w2_ref, b2_ref, g2_ref, be2_ref, eps_ref,
             ctrb_ref, wfc_ref, gf_ref, bef_ref, wf2t_ref, bf2_ref, out_ref):
        u = ((1.0 + eps_ref[0, 0]) * x1_ref[:n, :]
             + agg_ref[0, :n, :] + agg_ref[1, :n, :])
        y = jnp.dot(u, w2_ref[...], preferred_element_type=jnp.float32)
        y = y + b2_ref[...]
        x2 = _lrelu(_bn(y, g2_ref[...], be2_ref[...]))
        h = ctrb_ref[...] + jnp.dot(x2, wfc_ref[...],
                                    preferred_element_type=jnp.float32)
        h = _lrelu(_bn(h, gf_ref[...], bef_ref[...]))
        o = jnp.sum(h * wf2t_ref[...], axis=1, keepdims=True) + bf2_ref[0, 0]
        out_ref[...] = 1.0 / (1.0 + jnp.exp(-o))

    return pl.pallas_call(
        body, out_shape=jax.ShapeDtypeStruct((n, 1), jnp.float32))


def kernel(node_deg, node_lab, pos, edge_index, embed_deg, embed_lab,
           eps1, W1, b1, g1, be1, eps2, W2, b2, g2, be2,
           Wf1, bf1, gf, bef, Wf2, bf2):
    n = pos.shape[0]
    e = edge_index.shape[1]
    d = embed_deg.shape[1]
    np_ = n + 128

    epw = -(-e // (_NT * _CH)) * _CH
    if (epw // _CH) % 2:
        epw += _CH
    nchunk = epw // _CH
    ep = epw * _NT

    src = edge_index[0].astype(jnp.int32)
    dst = edge_index[1].astype(jnp.int32)
    sent = n + (jnp.arange(ep - e, dtype=jnp.int32) % 128)
    src_p = jnp.concatenate([src, sent]).reshape(_NT, epw)
    dst_p = jnp.concatenate([dst, sent]).reshape(_NT, nchunk, _CH)

    pos_p = jnp.zeros((np_, 16), jnp.float32).at[:n, :3].set(pos)
    zeros16 = jnp.zeros((np_, 16), jnp.float32)
    zeros_d = jnp.zeros((np_, d), jnp.float32)
    w1p = jnp.zeros((16, d), jnp.float32).at[:3, :].set(W1)
    ed_p = jnp.zeros((128, d), jnp.float32).at[:embed_deg.shape[0], :].set(
        embed_deg)
    el_p = jnp.zeros((128, d), jnp.float32).at[:embed_lab.shape[0], :].set(
        embed_lab)
    idx2 = jnp.stack([node_deg.astype(jnp.int32),
                      node_lab.astype(jnp.int32)], axis=1)
    row = lambda a: a.reshape(1, -1).astype(jnp.float32)
    sca = lambda a: a.reshape(1, 1).astype(jnp.float32)

    ctrb = _make_ctrb(n)(idx2, ed_p, el_p, Wf1[0:d], Wf1[d:2 * d], row(bf1))

    agg1 = _make_edge_agg(np_, epw, nchunk, 16)(pos_p, src_p, dst_p, zeros16)
    x1 = _make_layer1(n, np_)(pos_p, agg1, w1p, row(b1), row(g1), row(be1),
                              sca(eps1))
    agg2 = _make_edge_agg(np_, epw, nchunk, d)(x1, src_p, dst_p, zeros_d)
    out = _make_tail(n)(x1, agg2, W2, row(b2), row(g2), row(be2), sca(eps2),
                        ctrb, Wf1[2 * d:3 * d], row(gf), row(bef), row(Wf2.T),
                        sca(bf2))
    return out[:, 0]


# speedup vs baseline: 13.7077x; 1.2905x over previous
import functools

import jax
import jax.numpy as jnp
from jax import lax
from jax.experimental import pallas as pl
from jax.experimental.pallas import tpu as pltpu
from jax.experimental.pallas import tpu_sc as plsc

_NC = 2
_NS = 16
_NT = _NC * _NS
_CH = 80
_NB = 4


def _make_edge_agg(n, epw, nchunk, d, stage_x):
    mesh = plsc.VectorSubcoreMesh(core_axis_name="c", subcore_axis_name="s")

    scratch = (
        [pltpu.VMEM((_CH,), jnp.int32) for _ in range(_NB)]
        + [pltpu.VMEM((_CH,), jnp.int32) for _ in range(_NB)]
        + [pltpu.VMEM((_CH, d), jnp.float32) for _ in range(_NB)]
        + [pltpu.VMEM_SHARED((n, d), jnp.float32)]
        + ([pltpu.VMEM_SHARED((n, d), jnp.float32)] if stage_x else [])
        + [pltpu.SemaphoreType.DMA] * (4 * _NB)
    )

    @functools.partial(
        pl.kernel,
        out_type=jax.ShapeDtypeStruct((_NC, n, d), jnp.float32),
        mesh=mesh,
        compiler_params=pltpu.CompilerParams(use_tc_tiling_on_sc=False),
        scratch_types=scratch,
    )
    def agg(x_hbm, src_hbm, dst_hbm, zero_hbm, out_hbm, *refs):
        sidx = refs[0:_NB]
        didx = refs[_NB:2 * _NB]
        rows = refs[2 * _NB:3 * _NB]
        agg_sh = refs[3 * _NB]
        x_src = refs[3 * _NB + 1] if stage_x else x_hbm
        sems = refs[3 * _NB + 1 + (1 if stage_x else 0):]
        ssem = sems[0:_NB]
        dsem = sems[_NB:2 * _NB]
        gsem = sems[2 * _NB:3 * _NB]
        wsem = sems[3 * _NB:4 * _NB]

        c = lax.axis_index("c")
        s = lax.axis_index("s")
        wid = c * _NS + s

        @pl.when(s == 0)
        def _():
            pltpu.sync_copy(zero_hbm, agg_sh)
            if stage_x:
                pltpu.sync_copy(x_hbm, x_src)

        def iload(i):
            b = i % _NB
            return (pltpu.async_copy(src_hbm.at[wid, i], sidx[b], ssem[b]),
                    pltpu.async_copy(dst_hbm.at[wid, i], didx[b], dsem[b]))

        def gather(i):
            b = i % _NB
            return pltpu.async_copy(x_src.at[sidx[b]], rows[b], gsem[b])

        def scat(i):
            b = i % _NB
            return pltpu.async_copy(rows[b], agg_sh.at[didx[b]], wsem[b],
                                    add=True)

        idd = {i: iload(i) for i in range(min(3, nchunk))}
        plsc.subcore_barrier()

        gd = {}
        for i in range(min(2, nchunk)):
            idd[i][0].wait()
            gd[i] = gather(i)

        sd = {}
        for i in range(nchunk):
            gd[i].wait()
            idd[i][1].wait()
            sd[i] = scat(i)
            if i + 3 < nchunk:
                if i >= 1:
                    sd[i - 1].wait()
                idd[i + 3] = iload(i + 3)
            if i + 2 < nchunk:
                idd[i + 2][0].wait()
                gd[i + 2] = gather(i + 2)
        for i in range(max(0, nchunk - 4), nchunk):
            sd[i].wait()

        plsc.subcore_barrier()

        for cc in range(_NC):
            @pl.when((s == 0) & (c == cc))
            def _():
                pltpu.sync_copy(agg_sh, out_hbm.at[cc])

    return agg


def _lrelu(x):
    return jnp.where(x >= 0, x, 0.01 * x)


def _bn(y, g, b):
    m = jnp.mean(y, axis=0, keepdims=True)
    v = jnp.mean((y - m) * (y - m), axis=0, keepdims=True)
    return g * (y - m) * lax.rsqrt(v + 1e-5) + b


def _make_layer1(n):
    def body(pos_ref, agg_ref, w1_ref, b1_ref, g1_ref, be1_ref, eps_ref,
             idx_ref, ed_ref, el_ref, wfa_ref, wfb_ref, bf1_ref,
             x1_ref, ctrb_ref):
        u = ((1.0 + eps_ref[0, 0]) * pos_ref[...]
             + agg_ref[0] + agg_ref[1])
        y = jnp.dot(u, w1_ref[...], preferred_element_type=jnp.float32)
        y = y + b1_ref[...]
        x1_ref[...] = _lrelu(_bn(y, g1_ref[...], be1_ref[...]))

        it = lax.broadcasted_iota(jnp.int32, (n, 128), 1)
        ohd = (idx_ref[:, 0:1] == it).astype(jnp.float32)
        ohl = (idx_ref[:, 1:2] == it).astype(jnp.float32)
        edw = jnp.dot(ed_ref[...], wfa_ref[...],
                      preferred_element_type=jnp.float32)
        elw = jnp.dot(el_ref[...], wfb_ref[...],
                      preferred_element_type=jnp.float32)
        ctrb_ref[...] = (
            jnp.dot(ohd, edw, preferred_element_type=jnp.float32)
            + jnp.dot(ohl, elw, preferred_element_type=jnp.float32)
            + bf1_ref[...])

    return pl.pallas_call(
        body, out_shape=[jax.ShapeDtypeStruct((n, 128), jnp.float32),
                         jax.ShapeDtypeStruct((n, 128), jnp.float32)])


def _make_tail(n):
    def body(x1_ref, agg_ref, w2_ref, b2_ref, g2_ref, be2_ref, eps_ref,
             ctrb_ref, wfc_ref, gf_ref, bef_ref, wf2t_ref, bf2_ref, out_ref):
        u = ((1.0 + eps_ref[0, 0]) * x1_ref[...]
             + agg_ref[0] + agg_ref[1])
        y = jnp.dot(u, w2_ref[...], preferred_element_type=jnp.float32)
        y = y + b2_ref[...]
        x2 = _lrelu(_bn(y, g2_ref[...], be2_ref[...]))
        h = ctrb_ref[...] + jnp.dot(x2, wfc_ref[...],
                                    preferred_element_type=jnp.float32)
        h = _lrelu(_bn(h, gf_ref[...], bef_ref[...]))
        o = jnp.sum(h * wf2t_ref[...], axis=1, keepdims=True) + bf2_ref[0, 0]
        out_ref[...] = 1.0 / (1.0 + jnp.exp(-o))

    return pl.pallas_call(
        body, out_shape=jax.ShapeDtypeStruct((n, 1), jnp.float32))


def kernel(node_deg, node_lab, pos, edge_index, embed_deg, embed_lab,
           eps1, W1, b1, g1, be1, eps2, W2, b2, g2, be2,
           Wf1, bf1, gf, bef, Wf2, bf2):
    n = pos.shape[0]
    e = edge_index.shape[1]
    d = embed_deg.shape[1]
    epw = e // _NT
    nchunk = epw // _CH

    src_p = edge_index[0].astype(jnp.int32).reshape(_NT, nchunk, _CH)
    dst_p = edge_index[1].astype(jnp.int32).reshape(_NT, nchunk, _CH)

    pos_p = jnp.zeros((n, 16), jnp.float32).at[:, :3].set(pos)
    zeros16 = jnp.zeros((n, 16), jnp.float32)
    zeros_d = jnp.zeros((n, d), jnp.float32)
    w1p = jnp.zeros((16, d), jnp.float32).at[:3, :].set(W1)
    ed_p = jnp.zeros((128, d), jnp.float32).at[:embed_deg.shape[0], :].set(
        embed_deg)
    el_p = jnp.zeros((128, d), jnp.float32).at[:embed_lab.shape[0], :].set(
        embed_lab)
    idx2 = jnp.stack([node_deg.astype(jnp.int32),
                      node_lab.astype(jnp.int32)], axis=1)
    row = lambda a: a.reshape(1, -1).astype(jnp.float32)
    sca = lambda a: a.reshape(1, 1).astype(jnp.float32)

    agg1 = _make_edge_agg(n, epw, nchunk, 16, True)(
        pos_p, src_p, dst_p, zeros16)
    x1, ctrb = _make_layer1(n)(
        pos_p, agg1, w1p, row(b1), row(g1), row(be1), sca(eps1),
        idx2, ed_p, el_p, Wf1[0:d], Wf1[d:2 * d], row(bf1))
    agg2 = _make_edge_agg(n, epw, nchunk, d, False)(
        x1, src_p, dst_p, zeros_d)
    out = _make_tail(n)(x1, agg2, W2, row(b2), row(g2), row(be2), sca(eps2),
                        ctrb, Wf1[2 * d:3 * d], row(gf), row(bef), row(Wf2.T),
                        sca(bf2))
    return out[:, 0]


# speedup vs baseline: 14.8837x; 1.0858x over previous
import functools

import jax
import jax.numpy as jnp
from jax import lax
from jax.experimental import pallas as pl
from jax.experimental.pallas import tpu as pltpu
from jax.experimental.pallas import tpu_sc as plsc

_NC = 2
_NS = 16
_NT = _NC * _NS
_CH = 80
_NB = 4


def _make_edge_agg(n, epw, nchunk, d, stage_x):
    mesh = plsc.VectorSubcoreMesh(core_axis_name="c", subcore_axis_name="s")
    nblk = n // _CH
    bpt = -(-nblk // _NS)

    scratch = (
        [pltpu.VMEM((_CH,), jnp.int32) for _ in range(_NB)]
        + [pltpu.VMEM((_CH,), jnp.int32) for _ in range(_NB)]
        + [pltpu.VMEM((_CH, d), jnp.float32) for _ in range(_NB)]
        + [pltpu.VMEM_SHARED((n, d), jnp.float32)]
        + ([pltpu.VMEM_SHARED((n, d), jnp.float32)] if stage_x else [])
        + [pltpu.SemaphoreType.DMA] * (4 * _NB)
    )

    @functools.partial(
        pl.kernel,
        out_type=jax.ShapeDtypeStruct((_NC, n, d), jnp.float32),
        mesh=mesh,
        compiler_params=pltpu.CompilerParams(use_tc_tiling_on_sc=False),
        scratch_types=scratch,
    )
    def agg(x_hbm, edge_hbm, out_hbm, *refs):
        sidx = refs[0:_NB]
        didx = refs[_NB:2 * _NB]
        rows = refs[2 * _NB:3 * _NB]
        agg_sh = refs[3 * _NB]
        x_src = refs[3 * _NB + 1] if stage_x else x_hbm
        sems = refs[3 * _NB + 1 + (1 if stage_x else 0):]
        ssem = sems[0:_NB]
        dsem = sems[_NB:2 * _NB]
        gsem = sems[2 * _NB:3 * _NB]
        wsem = sems[3 * _NB:4 * _NB]

        c = lax.axis_index("c")
        s = lax.axis_index("s")
        wid = c * _NS + s

        def iload(i):
            b = i % _NB
            return (pltpu.async_copy(edge_hbm.at[0, wid, i], sidx[b], ssem[b]),
                    pltpu.async_copy(edge_hbm.at[1, wid, i], didx[b], dsem[b]))

        def gather(i):
            b = i % _NB
            return pltpu.async_copy(x_src.at[sidx[b]], rows[b], gsem[b])

        def scat(i):
            b = i % _NB
            return pltpu.async_copy(rows[b], agg_sh.at[didx[b]], wsem[b],
                                    add=True)

        idd = {i: iload(i) for i in range(min(3, nchunk))}

        def zrow(r, carry):
            for cc in range(d // 16):
                rows[0][r, pl.ds(cc * 16, 16)] = jnp.zeros((16,), jnp.float32)
            return carry
        lax.fori_loop(0, _CH, zrow, 0)
        for j in range(bpt):
            blk = j * _NS + s

            @pl.when(blk < nblk)
            def _():
                pltpu.sync_copy(rows[0], agg_sh.at[pl.ds(blk * _CH, _CH)])

        if stage_x:
            @pl.when(s == 0)
            def _():
                pltpu.sync_copy(x_hbm, x_src)

        plsc.subcore_barrier()

        gd = {}
        for i in range(min(2, nchunk)):
            idd[i][0].wait()
            gd[i] = gather(i)

        sd = {}
        for i in range(nchunk):
            gd[i].wait()
            idd[i][1].wait()
            sd[i] = scat(i)
            if i + 3 < nchunk:
                if i >= 1:
                    sd[i - 1].wait()
                idd[i + 3] = iload(i + 3)
            if i + 2 < nchunk:
                idd[i + 2][0].wait()
                gd[i + 2] = gather(i + 2)
        for i in range(max(0, nchunk - 4), nchunk):
            sd[i].wait()

        plsc.subcore_barrier()

        for cc in range(_NC):
            @pl.when((s == 0) & (c == cc))
            def _():
                pltpu.sync_copy(agg_sh, out_hbm.at[cc])

    return agg


def _lrelu(x):
    return jnp.where(x >= 0, x, 0.01 * x)


def _bn(y, g, b):
    m = jnp.mean(y, axis=0, keepdims=True)
    v = jnp.mean((y - m) * (y - m), axis=0, keepdims=True)
    return g * (y - m) * lax.rsqrt(v + 1e-5) + b


def _make_ctrb(n):
    def body(idx_ref, ed_ref, el_ref, wf1_ref, bf1_ref, out_ref):
        d = 128
        it = lax.broadcasted_iota(jnp.int32, (n, 128), 1)
        ohd = (idx_ref[:, 0:1] == it).astype(jnp.float32)
        ohl = (idx_ref[:, 1:2] == it).astype(jnp.float32)
        ed_p = jnp.concatenate(
            [ed_ref[...], jnp.zeros((128 - ed_ref.shape[0], d), jnp.float32)],
            axis=0)
        el_p = jnp.concatenate(
            [el_ref[...], jnp.zeros((128 - el_ref.shape[0], d), jnp.float32)],
            axis=0)
        edw = jnp.dot(ed_p, wf1_ref[0:d], preferred_element_type=jnp.float32)
        elw = jnp.dot(el_p, wf1_ref[d:2 * d],
                      preferred_element_type=jnp.float32)
        out_ref[...] = (
            jnp.dot(ohd, edw, preferred_element_type=jnp.float32)
            + jnp.dot(ohl, elw, preferred_element_type=jnp.float32)
            + bf1_ref[...])

    return pl.pallas_call(
        body, out_shape=jax.ShapeDtypeStruct((n, 128), jnp.float32))


def _make_layer1(n):
    def body(pos_ref, agg_ref, w1_ref, b1_ref, g1_ref, be1_ref, eps_ref,
             x1_ref):
        u = ((1.0 + eps_ref[0, 0]) * pos_ref[...]
             + agg_ref[0] + agg_ref[1])
        w1p = jnp.concatenate(
            [w1_ref[...], jnp.zeros((13, 128), jnp.float32)], axis=0)
        y = jnp.dot(u, w1p, preferred_element_type=jnp.float32)
        y = y + b1_ref[...]
        x1_ref[...] = _lrelu(_bn(y, g1_ref[...], be1_ref[...]))

    return pl.pallas_call(
        body, out_shape=jax.ShapeDtypeStruct((n, 128), jnp.float32))


def _make_tail(n):
    def body(x1_ref, agg_ref, w2_ref, b2_ref, g2_ref, be2_ref, eps_ref,
             ctrb_ref, wf1_ref, gf_ref, bef_ref, wf2t_ref, bf2_ref, out_ref):
        u = ((1.0 + eps_ref[0, 0]) * x1_ref[...]
             + agg_ref[0] + agg_ref[1])
        y = jnp.dot(u, w2_ref[...], preferred_element_type=jnp.float32)
        y = y + b2_ref[...]
        x2 = _lrelu(_bn(y, g2_ref[...], be2_ref[...]))
        h = ctrb_ref[...] + jnp.dot(x2, wf1_ref[256:384],
                                    preferred_element_type=jnp.float32)
        h = _lrelu(_bn(h, gf_ref[...], bef_ref[...]))
        o = jnp.sum(h * wf2t_ref[...], axis=1, keepdims=True) + bf2_ref[0, 0]
        out_ref[...] = 1.0 / (1.0 + jnp.exp(-o))

    return pl.pallas_call(
        body, out_shape=jax.ShapeDtypeStruct((n, 1), jnp.float32))


def kernel(node_deg, node_lab, pos, edge_index, embed_deg, embed_lab,
           eps1, W1, b1, g1, be1, eps2, W2, b2, g2, be2,
           Wf1, bf1, gf, bef, Wf2, bf2):
    n = pos.shape[0]
    e = edge_index.shape[1]
    d = embed_deg.shape[1]
    epw = e // _NT
    nchunk = epw // _CH

    edge_r = edge_index.astype(jnp.int32).reshape(2, _NT, nchunk, _CH)
    pos_p = jnp.zeros((n, 16), jnp.float32).at[:, :3].set(pos)
    idx2 = jnp.stack([node_deg.astype(jnp.int32),
                      node_lab.astype(jnp.int32)], axis=1)
    row = lambda a: a.reshape(1, -1).astype(jnp.float32)
    sca = lambda a: a.reshape(1, 1).astype(jnp.float32)

    ctrb = _make_ctrb(n)(idx2, embed_deg, embed_lab, Wf1, row(bf1))
    agg1 = _make_edge_agg(n, epw, nchunk, 16, True)(pos_p, edge_r)
    x1 = _make_layer1(n)(pos_p, agg1, W1, row(b1), row(g1), row(be1),
                         sca(eps1))
    agg2 = _make_edge_agg(n, epw, nchunk, d, False)(x1, edge_r)
    out = _make_tail(n)(x1, agg2, W2, row(b2), row(g2), row(be2), sca(eps2),
                        ctrb, Wf1, row(gf), row(bef), row(Wf2.T), sca(bf2))
    return out[:, 0]
